# baseline (speedup 1.0000x reference)
import jax
import jax.numpy as jnp
from jax import lax
from jax.experimental import pallas as pl
from jax.experimental.pallas import tpu as pltpu
from jax.experimental.pallas import tpu_sc as plsc

N = 10000
E = 320000
FEAT = 128
NP = 10240
NC = 2
NT = 16
NW = NC * NT
BB = 128
TOTB = 2560
EP = TOTB * BB
GROUPS = BB // 16
DVT = NP // NT
VREGS = FEAT // 16
CH = 16
FAST_C = 0
NBT_F = 96
NBT_S = 64


def _hop_range(c, s):
    fast = c == FAST_C
    start = jnp.where(fast, s * NBT_F, NT * NBT_F + s * NBT_S)
    nch = jnp.where(fast, NBT_F // CH, NBT_S // CH)
    return start, nch


def _splat(v16, j):
    return v16.at[jnp.full((16,), j, jnp.int32)].get(mode="promise_in_bounds")


def _zero_shared(acc_sh, rows0, r0):
    z16 = jnp.zeros((16,), jnp.float32)
    for e in range(BB):
        for q in range(VREGS):
            rows0[e, pl.ds(q * 16, 16)] = z16
    for i in range(NP // NT // BB):
        pltpu.sync_copy(rows0, acc_sh.at[pl.ds(r0 + i * BB, BB)])


def _scale_rows(rows_p, normch, row):
    for g in range(GROUPS):
        n16 = normch[row, pl.ds(g * 16, 16)]
        for j in range(16):
            sc = _splat(n16, j)
            e = g * 16 + j
            for q in range(VREGS):
                rows_p[e, pl.ds(q * 16, 16)] = rows_p[e, pl.ds(q * 16, 16)] * sc


def _phase(k, p, t, last_pair, table, srcc, dstc, normch, rows, gsems, ssems,
           acc_sh):
    rows_p, rows_q = rows[p], rows[1 - p]
    pltpu.make_async_copy(table.at[pl.ds(0, BB)], rows_p, gsems[p]).wait()
    _scale_rows(rows_p, normch, k)

    if p == 0:
        @pl.when(t > 0)
        def _():
            pltpu.make_async_copy(rows_q, acc_sh.at[pl.ds(0, BB)],
                                  ssems[1 - p]).wait()
        pltpu.async_copy(table.at[srcc.at[k + 1]], rows_q, gsems[1 - p])
    else:
        pltpu.make_async_copy(rows_q, acc_sh.at[pl.ds(0, BB)],
                              ssems[1 - p]).wait()

        @pl.when(jnp.logical_not(last_pair))
        def _():
            pltpu.async_copy(table.at[srcc.at[k + 1]], rows_q, gsems[1 - p])

    pltpu.async_copy(rows_p, acc_sh.at[dstc.at[k]], ssems[p], add=True)


def _deg_body(ei3, ew2, deg_out, deg_sh, srcc, ewc, ewd, degloc, sem):
    c = lax.axis_index("c")
    s = lax.axis_index("s")
    wid = s * NC + c
    z16 = jnp.zeros((16,), jnp.float32)
    for i in range(DVT // 16):
        degloc[pl.ds(i * 16, 16)] = z16
    pltpu.sync_copy(degloc, deg_sh.at[pl.ds(s * DVT, DVT)])
    plsc.subcore_barrier()

    start = wid * (TOTB // NW)

    def chunk(ch, carry):
        cb = start + ch * CH
        pltpu.sync_copy(ei3.at[0].at[pl.ds(cb, CH)], srcc)
        pltpu.sync_copy(ew2.at[pl.ds(cb, CH)], ewc)
        for r in range(CH):
            pltpu.async_copy(ewc.at[r], deg_sh.at[srcc.at[r]], sem, add=True)
        for r in range(CH):
            pltpu.make_async_copy(ew2.at[0], ewd, sem).wait()
        return carry

    lax.fori_loop(0, (TOTB // NW) // CH, chunk, 0)
    plsc.subcore_barrier()
    pltpu.sync_copy(deg_sh.at[pl.ds(s * DVT, DVT)],
                    deg_out.at[c].at[pl.ds(s * DVT, DVT)])


def _hop1_body(ei3, ew2, x, dinv, tx1_out, norm_out,
               tx1_sh, dinv_sh, rows0, rows1, srcc, dstc, normch, dvs, dvd,
               gsem0, gsem1, nsem, ssem0, ssem1):
    c = lax.axis_index("c")
    s = lax.axis_index("s")
    wid = s * NC + c
    r0 = s * (NP // NT)

    _zero_shared(tx1_sh, rows0, r0)

    @pl.when(s == 0)
    def _():
        pltpu.sync_copy(dinv, dinv_sh)
    plsc.subcore_barrier()

    start, nch = _hop_range(c, s)

    def chunk(ch, carry):
        cb = start + ch * CH
        pltpu.sync_copy(ei3.at[0].at[pl.ds(cb, CH)], srcc)
        pltpu.sync_copy(ei3.at[1].at[pl.ds(cb, CH)], dstc)
        pltpu.sync_copy(ew2.at[pl.ds(cb, CH)], normch)

        for r in range(CH):
            pltpu.async_copy(dinv_sh.at[srcc.at[r]], dvs.at[r], nsem)
            pltpu.async_copy(dinv_sh.at[dstc.at[r]], dvd.at[r], nsem)
        for r in range(2 * CH):
            pltpu.make_async_copy(dinv_sh.at[pl.ds(0, BB)], dvs.at[0],
                                  nsem).wait()

        def normrow(r, cc):
            for g in range(GROUPS):
                sl = pl.ds(g * 16, 16)
                normch[r, sl] = -(dvs[r, sl] * normch[r, sl] * dvd[r, sl])
            return cc

        lax.fori_loop(0, CH, normrow, 0)
        pltpu.sync_copy(normch, norm_out.at[pl.ds(cb, CH)])

        def step(kk, cc):
            even = (kk & 1) == 0

            @pl.when(jnp.logical_and(even, kk < CH))
            def _():
                @pl.when(kk >= 2)
                def _():
                    pltpu.make_async_copy(rows0, tx1_sh.at[pl.ds(0, BB)],
                                          ssem0).wait()
                pltpu.async_copy(x.at[srcc.at[kk]], rows0, gsem0)

            @pl.when(jnp.logical_and(jnp.logical_not(even), kk < CH))
            def _():
                @pl.when(kk >= 2)
                def _():
                    pltpu.make_async_copy(rows1, tx1_sh.at[pl.ds(0, BB)],
                                          ssem1).wait()
                pltpu.async_copy(x.at[srcc.at[kk]], rows1, gsem1)

            @pl.when(jnp.logical_and(jnp.logical_not(even), kk >= 1))
            def _():
                pltpu.make_async_copy(x.at[pl.ds(0, BB)], rows0,
                                      gsem0).wait()
                _scale_rows(rows0, normch, kk - 1)
                pltpu.async_copy(rows0, tx1_sh.at[dstc.at[kk - 1]], ssem0,
                                 add=True)

            @pl.when(jnp.logical_and(even, kk >= 1))
            def _():
                pltpu.make_async_copy(x.at[pl.ds(0, BB)], rows1,
                                      gsem1).wait()
                _scale_rows(rows1, normch, kk - 1)
                pltpu.async_copy(rows1, tx1_sh.at[dstc.at[kk - 1]], ssem1,
                                 add=True)

            return cc

        lax.fori_loop(0, CH + 1, step, 0)
        pltpu.make_async_copy(rows0, tx1_sh.at[pl.ds(0, BB)], ssem0).wait()
        pltpu.make_async_copy(rows1, tx1_sh.at[pl.ds(0, BB)], ssem1).wait()
        return carry

    lax.fori_loop(0, nch, chunk, 0)
    plsc.subcore_barrier()

    pltpu.sync_copy(tx1_sh.at[pl.ds(r0, NP // NT)],
                    tx1_out.at[c].at[pl.ds(r0, NP // NT)])


def _hop2_body(ei3, nrm2, tx1, s2_out,
               s2_sh, rows0, rows1, srcc, dstc, normch,
               gsem0, gsem1, ssem0, ssem1):
    c = lax.axis_index("c")
    s = lax.axis_index("s")
    wid = s * NC + c
    r0 = s * (NP // NT)
    _zero_shared(s2_sh, rows0, r0)
    plsc.subcore_barrier()

    start, nch = _hop_range(c, s)

    def chunk(ch, carry):
        cb = start + ch * CH
        pltpu.sync_copy(ei3.at[0].at[pl.ds(cb, CH)], srcc)
        pltpu.sync_copy(ei3.at[1].at[pl.ds(cb, CH)], dstc)
        pltpu.sync_copy(nrm2.at[pl.ds(cb, CH)], normch)

        def step(kk, cc):
            even = (kk & 1) == 0

            @pl.when(jnp.logical_and(even, kk < CH))
            def _():
                @pl.when(kk >= 2)
                def _():
                    pltpu.make_async_copy(rows0, s2_sh.at[pl.ds(0, BB)],
                                          ssem0).wait()
                pltpu.async_copy(tx1.at[srcc.at[kk]], rows0, gsem0)

            @pl.when(jnp.logical_and(jnp.logical_not(even), kk < CH))
            def _():
                @pl.when(kk >= 2)
                def _():
                    pltpu.make_async_copy(rows1, s2_sh.at[pl.ds(0, BB)],
                                          ssem1).wait()
                pltpu.async_copy(tx1.at[srcc.at[kk]], rows1, gsem1)

            @pl.when(jnp.logical_and(jnp.logical_not(even), kk >= 1))
            def _():
                pltpu.make_async_copy(tx1.at[pl.ds(0, BB)], rows0,
                                      gsem0).wait()
                _scale_rows(rows0, normch, kk - 1)
                pltpu.async_copy(rows0, s2_sh.at[dstc.at[kk - 1]], ssem0,
                                 add=True)

            @pl.when(jnp.logical_and(even, kk >= 1))
            def _():
                pltpu.make_async_copy(tx1.at[pl.ds(0, BB)], rows1,
                                      gsem1).wait()
                _scale_rows(rows1, normch, kk - 1)
                pltpu.async_copy(rows1, s2_sh.at[dstc.at[kk - 1]], ssem1,
                                 add=True)

            return cc

        lax.fori_loop(0, CH + 1, step, 0)
        pltpu.make_async_copy(rows0, s2_sh.at[pl.ds(0, BB)], ssem0).wait()
        pltpu.make_async_copy(rows1, s2_sh.at[pl.ds(0, BB)], ssem1).wait()
        return carry

    lax.fori_loop(0, nch, chunk, 0)
    plsc.subcore_barrier()

    pltpu.sync_copy(s2_sh.at[pl.ds(r0, NP // NT)],
                    s2_out.at[c].at[pl.ds(r0, NP // NT)])


def _sc_mesh():
    return plsc.VectorSubcoreMesh(core_axis_name="c", subcore_axis_name="s")


def _deg(ei3, ew2):
    f32 = jnp.float32
    kern = pl.kernel(
        _deg_body,
        out_type=[jax.ShapeDtypeStruct((NC, NP), f32)],
        mesh=_sc_mesh(),
        compiler_params=pltpu.CompilerParams(needs_layout_passes=False),
        scratch_types=[
            pltpu.VMEM_SHARED((NP,), f32),
            pltpu.VMEM((CH, BB), jnp.int32),
            pltpu.VMEM((CH, BB), f32),
            pltpu.VMEM((BB,), f32),
            pltpu.VMEM((DVT,), f32),
            pltpu.SemaphoreType.DMA,
        ],
    )
    return kern(ei3, ew2)[0]


def _hop1(ei3, ew2, x, dinv):
    f32 = jnp.float32
    kern = pl.kernel(
        _hop1_body,
        out_type=[jax.ShapeDtypeStruct((NC, NP, FEAT), f32),
                  jax.ShapeDtypeStruct((TOTB, BB), f32)],
        mesh=_sc_mesh(),
        compiler_params=pltpu.CompilerParams(needs_layout_passes=False),
        scratch_types=[
            pltpu.VMEM_SHARED((NP, FEAT), f32),
            pltpu.VMEM_SHARED((NP,), f32),
            pltpu.VMEM((BB, FEAT), f32),
            pltpu.VMEM((BB, FEAT), f32),
            pltpu.VMEM((CH, BB), jnp.int32),
            pltpu.VMEM((CH, BB), jnp.int32),
            pltpu.VMEM((CH, BB), f32),
            pltpu.VMEM((CH, BB), f32),
            pltpu.VMEM((CH, BB), f32),
            pltpu.SemaphoreType.DMA,
            pltpu.SemaphoreType.DMA,
            pltpu.SemaphoreType.DMA,
            pltpu.SemaphoreType.DMA,
            pltpu.SemaphoreType.DMA,
        ],
    )
    return kern(ei3, ew2, x, dinv)


def _hop2(ei3, nrm2, tx1):
    f32 = jnp.float32
    kern = pl.kernel(
        _hop2_body,
        out_type=[jax.ShapeDtypeStruct((NC, NP, FEAT), f32)],
        mesh=_sc_mesh(),
        compiler_params=pltpu.CompilerParams(needs_layout_passes=False),
        scratch_types=[
            pltpu.VMEM_SHARED((NP, FEAT), f32),
            pltpu.VMEM((BB, FEAT), f32),
            pltpu.VMEM((BB, FEAT), f32),
            pltpu.VMEM((CH, BB), jnp.int32),
            pltpu.VMEM((CH, BB), jnp.int32),
            pltpu.VMEM((CH, BB), f32),
            pltpu.SemaphoreType.DMA,
            pltpu.SemaphoreType.DMA,
            pltpu.SemaphoreType.DMA,
            pltpu.SemaphoreType.DMA,
        ],
    )
    return kern(ei3, nrm2, tx1)[0]


def _dinv_body(degp_ref, out_ref):
    d = degp_ref[0] + degp_ref[1]
    out_ref[...] = jnp.where(d > 0, lax.rsqrt(jnp.maximum(d, 1e-12)), 0.0)


def _dinv(degp):
    return pl.pallas_call(
        _dinv_body,
        out_shape=jax.ShapeDtypeStruct((NP // FEAT, FEAT), jnp.float32),
    )(degp.reshape(NC, NP // FEAT, FEAT)).reshape(NP)


def _sum_body(p_ref, out_ref):
    out_ref[...] = p_ref[0] + p_ref[1]


def _sum_partials(p):
    R = 1024
    return pl.pallas_call(
        _sum_body,
        grid=(NP // R,),
        in_specs=[pl.BlockSpec((NC, R, FEAT), lambda i: (0, i, 0))],
        out_specs=pl.BlockSpec((R, FEAT), lambda i: (i, 0)),
        out_shape=jax.ShapeDtypeStruct((NP, FEAT), jnp.float32),
    )(p)


def _dense_body(x_ref, t1_ref, s2_ref, wzh_ref, bzh_ref, wlin_ref, blin_ref,
                out_ref):
    xb = x_ref[...]
    t1 = t1_ref[...]
    s2 = s2_ref[0] + s2_ref[1]
    tx2 = 2.0 * s2 - xb
    xt = jnp.concatenate([xb, t1, tx2], axis=1)
    a = jnp.dot(xt, wzh_ref[...], preferred_element_type=jnp.float32)
    a = a + bzh_ref[...]
    z = jax.nn.sigmoid(a[:, :FEAT])
    ht = jnp.tanh(a[:, FEAT:])
    h = jnp.maximum((1.0 - z) * ht, 0.0)
    out_ref[...] = (jnp.dot(h, wlin_ref[...], preferred_element_type=jnp.float32)
                    + blin_ref[...])


def _dense(x, tx1, s2p, wzh, bzh, wlin, blin):
    R = 512
    return pl.pallas_call(
        _dense_body,
        grid=(NP // R,),
        in_specs=[
            pl.BlockSpec((R, FEAT), lambda i: (i, 0)),
            pl.BlockSpec((R, FEAT), lambda i: (i, 0)),
            pl.BlockSpec((NC, R, FEAT), lambda i: (0, i, 0)),
            pl.BlockSpec((3 * FEAT, 2 * FEAT), lambda i: (0, 0)),
            pl.BlockSpec((1, 2 * FEAT), lambda i: (0, 0)),
            pl.BlockSpec((FEAT, FEAT), lambda i: (0, 0)),
            pl.BlockSpec((1, FEAT), lambda i: (0, 0)),
        ],
        out_specs=pl.BlockSpec((R, FEAT), lambda i: (i, 0)),
        out_shape=jax.ShapeDtypeStruct((N, FEAT), jnp.float32),
    )(x, tx1, s2p, wzh, bzh, wlin, blin)


def kernel(x, edge_index, edge_weight, Wxz, bxz, Whz, bhz, Wxr, bxr, Whr, bhr,
           Wxh, bxh, Whh, bhh, Wlin, blin):
    pad = EP - E
    ei3 = jnp.concatenate(
        [edge_index, jnp.zeros((2, pad), jnp.int32)], axis=1
    ).reshape(2, TOTB, BB)
    ew2 = jnp.concatenate(
        [edge_weight, jnp.zeros((pad,), jnp.float32)]).reshape(TOTB, BB)
    degp = _deg(ei3, ew2)
    dinv = _dinv(degp)
    tx1p, nrm2 = _hop1(ei3, ew2, x, dinv)
    tx1 = _sum_partials(tx1p)
    s2p = _hop2(ei3, nrm2, tx1)
    wzh = jnp.concatenate([Wxz.reshape(3 * FEAT, FEAT),
                           Wxh.reshape(3 * FEAT, FEAT)], axis=1)
    bzh = jnp.concatenate([bxz + bhz, bxh + bhh]).reshape(1, 2 * FEAT)
    return _dense(x, tx1[:N], s2p[:, :N], wzh, bzh, Wlin, blin.reshape(1, FEAT))


# speedup vs baseline: 1.0328x; 1.0328x over previous
import jax
import jax.numpy as jnp
from jax import lax
from jax.experimental import pallas as pl
from jax.experimental.pallas import tpu as pltpu
from jax.experimental.pallas import tpu_sc as plsc

N = 10000
E = 320000
FEAT = 128
NP = 10240
NC = 2
NT = 16
NW = NC * NT
BB = 128
TOTB = 2560
EP = TOTB * BB
GROUPS = BB // 16
DVT = NP // NT
VREGS = FEAT // 16
CH = 16
FAST_C = 0
NBT_F = 112
NBT_S = 48


def _hop_range(c, s):
    fast = c == FAST_C
    start = jnp.where(fast, s * NBT_F, NT * NBT_F + s * NBT_S)
    nch = jnp.where(fast, NBT_F // CH, NBT_S // CH)
    return start, nch


def _rsqrt16(v):
    bits = lax.bitcast_convert_type(v, jnp.int32)
    y = lax.bitcast_convert_type(
        jnp.full((16,), 0x5F3759DF, jnp.int32) - (bits >> 1), jnp.float32)
    for _ in range(3):
        y = y * (1.5 - 0.5 * v * y * y)
    return y


def _splat(v16, j):
    return v16.at[jnp.full((16,), j, jnp.int32)].get(mode="promise_in_bounds")


def _zero_shared(acc_sh, rows0, r0):
    z16 = jnp.zeros((16,), jnp.float32)
    for e in range(BB):
        for q in range(VREGS):
            rows0[e, pl.ds(q * 16, 16)] = z16
    for i in range(NP // NT // BB):
        pltpu.sync_copy(rows0, acc_sh.at[pl.ds(r0 + i * BB, BB)])


def _scale_rows(rows_p, normch, row):
    for g in range(GROUPS):
        n16 = normch[row, pl.ds(g * 16, 16)]
        for j in range(16):
            sc = _splat(n16, j)
            e = g * 16 + j
            for q in range(VREGS):
                rows_p[e, pl.ds(q * 16, 16)] = rows_p[e, pl.ds(q * 16, 16)] * sc


def _phase(k, p, t, last_pair, table, srcc, dstc, normch, rows, gsems, ssems,
           acc_sh):
    rows_p, rows_q = rows[p], rows[1 - p]
    pltpu.make_async_copy(table.at[pl.ds(0, BB)], rows_p, gsems[p]).wait()
    _scale_rows(rows_p, normch, k)

    if p == 0:
        @pl.when(t > 0)
        def _():
            pltpu.make_async_copy(rows_q, acc_sh.at[pl.ds(0, BB)],
                                  ssems[1 - p]).wait()
        pltpu.async_copy(table.at[srcc.at[k + 1]], rows_q, gsems[1 - p])
    else:
        pltpu.make_async_copy(rows_q, acc_sh.at[pl.ds(0, BB)],
                              ssems[1 - p]).wait()

        @pl.when(jnp.logical_not(last_pair))
        def _():
            pltpu.async_copy(table.at[srcc.at[k + 1]], rows_q, gsems[1 - p])

    pltpu.async_copy(rows_p, acc_sh.at[dstc.at[k]], ssems[p], add=True)


def _deg_body(ei3, ew2, deg_out, deg_sh, srcc, ewc, ewd, degloc, sem):
    c = lax.axis_index("c")
    s = lax.axis_index("s")
    wid = s * NC + c
    z16 = jnp.zeros((16,), jnp.float32)
    for i in range(DVT // 16):
        degloc[pl.ds(i * 16, 16)] = z16
    pltpu.sync_copy(degloc, deg_sh.at[pl.ds(s * DVT, DVT)])
    plsc.subcore_barrier()

    start = wid * (TOTB // NW)

    def chunk(ch, carry):
        cb = start + ch * CH
        pltpu.sync_copy(ei3.at[0].at[pl.ds(cb, CH)], srcc)
        pltpu.sync_copy(ew2.at[pl.ds(cb, CH)], ewc)
        for r in range(CH):
            pltpu.async_copy(ewc.at[r], deg_sh.at[srcc.at[r]], sem, add=True)
        for r in range(CH):
            pltpu.make_async_copy(ew2.at[0], ewd, sem).wait()
        return carry

    lax.fori_loop(0, (TOTB // NW) // CH, chunk, 0)
    plsc.subcore_barrier()
    pltpu.sync_copy(deg_sh.at[pl.ds(s * DVT, DVT)],
                    deg_out.at[c].at[pl.ds(s * DVT, DVT)])


def _hop1_body(ei3, ew2, x, tx1_out, norm_out,
               tx1_sh, dinv_sh, rows0, rows1, srcc, dstc, normch, dvs, dvd,
               degloc, gsem0, gsem1, nsem, ssem0, ssem1):
    c = lax.axis_index("c")
    s = lax.axis_index("s")
    wid = s * NC + c
    r0 = s * (NP // NT)

    _zero_shared(tx1_sh, rows0, r0)
    z16 = jnp.zeros((16,), jnp.float32)
    for i in range(DVT // 16):
        degloc[pl.ds(i * 16, 16)] = z16
    pltpu.sync_copy(degloc, dinv_sh.at[pl.ds(s * DVT, DVT)])
    plsc.subcore_barrier()

    def deg_chunk(ch, carry):
        cb = (s + ch * NT) * CH
        pltpu.sync_copy(ei3.at[0].at[pl.ds(cb, CH)], srcc)
        pltpu.sync_copy(ew2.at[pl.ds(cb, CH)], normch)
        for r in range(CH):
            pltpu.async_copy(normch.at[r], dinv_sh.at[srcc.at[r]], nsem,
                             add=True)
        for r in range(CH):
            pltpu.make_async_copy(ew2.at[0], dvd.at[0], nsem).wait()
        return carry

    lax.fori_loop(0, TOTB // CH // NT, deg_chunk, 0)
    plsc.subcore_barrier()

    pltpu.sync_copy(dinv_sh.at[pl.ds(s * DVT, DVT)], degloc)
    for i in range(DVT // 16):
        d16 = degloc[pl.ds(i * 16, 16)]
        y = _rsqrt16(jnp.maximum(d16, 1e-12))
        degloc[pl.ds(i * 16, 16)] = jnp.where(d16 > 0, y, 0.0)
    pltpu.sync_copy(degloc, dinv_sh.at[pl.ds(s * DVT, DVT)])
    plsc.subcore_barrier()

    start, nch = _hop_range(c, s)

    def chunk(ch, carry):
        cb = start + ch * CH
        pltpu.sync_copy(ei3.at[0].at[pl.ds(cb, CH)], srcc)
        pltpu.sync_copy(ei3.at[1].at[pl.ds(cb, CH)], dstc)
        pltpu.sync_copy(ew2.at[pl.ds(cb, CH)], normch)

        for r in range(CH):
            pltpu.async_copy(dinv_sh.at[srcc.at[r]], dvs.at[r], nsem)
            pltpu.async_copy(dinv_sh.at[dstc.at[r]], dvd.at[r], nsem)
        for r in range(2 * CH):
            pltpu.make_async_copy(dinv_sh.at[pl.ds(0, BB)], dvs.at[0],
                                  nsem).wait()

        def normrow(r, cc):
            for g in range(GROUPS):
                sl = pl.ds(g * 16, 16)
                normch[r, sl] = -(dvs[r, sl] * normch[r, sl] * dvd[r, sl])
            return cc

        lax.fori_loop(0, CH, normrow, 0)
        pltpu.sync_copy(normch, norm_out.at[pl.ds(cb, CH)])

        def step(kk, cc):
            even = (kk & 1) == 0

            @pl.when(jnp.logical_and(even, kk < CH))
            def _():
                @pl.when(kk >= 2)
                def _():
                    pltpu.make_async_copy(rows0, tx1_sh.at[pl.ds(0, BB)],
                                          ssem0).wait()
                pltpu.async_copy(x.at[srcc.at[kk]], rows0, gsem0)

            @pl.when(jnp.logical_and(jnp.logical_not(even), kk < CH))
            def _():
                @pl.when(kk >= 2)
                def _():
                    pltpu.make_async_copy(rows1, tx1_sh.at[pl.ds(0, BB)],
                                          ssem1).wait()
                pltpu.async_copy(x.at[srcc.at[kk]], rows1, gsem1)

            @pl.when(jnp.logical_and(jnp.logical_not(even), kk >= 1))
            def _():
                pltpu.make_async_copy(x.at[pl.ds(0, BB)], rows0,
                                      gsem0).wait()
                _scale_rows(rows0, normch, kk - 1)
                pltpu.async_copy(rows0, tx1_sh.at[dstc.at[kk - 1]], ssem0,
                                 add=True)

            @pl.when(jnp.logical_and(even, kk >= 1))
            def _():
                pltpu.make_async_copy(x.at[pl.ds(0, BB)], rows1,
                                      gsem1).wait()
                _scale_rows(rows1, normch, kk - 1)
                pltpu.async_copy(rows1, tx1_sh.at[dstc.at[kk - 1]], ssem1,
                                 add=True)

            return cc

        lax.fori_loop(0, CH + 1, step, 0)
        pltpu.make_async_copy(rows0, tx1_sh.at[pl.ds(0, BB)], ssem0).wait()
        pltpu.make_async_copy(rows1, tx1_sh.at[pl.ds(0, BB)], ssem1).wait()
        return carry

    lax.fori_loop(0, nch, chunk, 0)
    plsc.subcore_barrier()

    pltpu.sync_copy(tx1_sh.at[pl.ds(r0, NP // NT)],
                    tx1_out.at[c].at[pl.ds(r0, NP // NT)])


def _hop2_body(ei3, nrm2, tx1, s2_out,
               s2_sh, rows0, rows1, srcc, dstc, normch,
               gsem0, gsem1, ssem0, ssem1):
    c = lax.axis_index("c")
    s = lax.axis_index("s")
    wid = s * NC + c
    r0 = s * (NP // NT)
    _zero_shared(s2_sh, rows0, r0)
    plsc.subcore_barrier()

    start, nch = _hop_range(c, s)

    def chunk(ch, carry):
        cb = start + ch * CH
        pltpu.sync_copy(ei3.at[0].at[pl.ds(cb, CH)], srcc)
        pltpu.sync_copy(ei3.at[1].at[pl.ds(cb, CH)], dstc)
        pltpu.sync_copy(nrm2.at[pl.ds(cb, CH)], normch)

        def step(kk, cc):
            even = (kk & 1) == 0

            @pl.when(jnp.logical_and(even, kk < CH))
            def _():
                @pl.when(kk >= 2)
                def _():
                    pltpu.make_async_copy(rows0, s2_sh.at[pl.ds(0, BB)],
                                          ssem0).wait()
                pltpu.async_copy(tx1.at[srcc.at[kk]], rows0, gsem0)

            @pl.when(jnp.logical_and(jnp.logical_not(even), kk < CH))
            def _():
                @pl.when(kk >= 2)
                def _():
                    pltpu.make_async_copy(rows1, s2_sh.at[pl.ds(0, BB)],
                                          ssem1).wait()
                pltpu.async_copy(tx1.at[srcc.at[kk]], rows1, gsem1)

            @pl.when(jnp.logical_and(jnp.logical_not(even), kk >= 1))
            def _():
                pltpu.make_async_copy(tx1.at[pl.ds(0, BB)], rows0,
                                      gsem0).wait()
                _scale_rows(rows0, normch, kk - 1)
                pltpu.async_copy(rows0, s2_sh.at[dstc.at[kk - 1]], ssem0,
                                 add=True)

            @pl.when(jnp.logical_and(even, kk >= 1))
            def _():
                pltpu.make_async_copy(tx1.at[pl.ds(0, BB)], rows1,
                                      gsem1).wait()
                _scale_rows(rows1, normch, kk - 1)
                pltpu.async_copy(rows1, s2_sh.at[dstc.at[kk - 1]], ssem1,
                                 add=True)

            return cc

        lax.fori_loop(0, CH + 1, step, 0)
        pltpu.make_async_copy(rows0, s2_sh.at[pl.ds(0, BB)], ssem0).wait()
        pltpu.make_async_copy(rows1, s2_sh.at[pl.ds(0, BB)], ssem1).wait()
        return carry

    lax.fori_loop(0, nch, chunk, 0)
    plsc.subcore_barrier()

    pltpu.sync_copy(s2_sh.at[pl.ds(r0, NP // NT)],
                    s2_out.at[c].at[pl.ds(r0, NP // NT)])


def _sc_mesh():
    return plsc.VectorSubcoreMesh(core_axis_name="c", subcore_axis_name="s")


def _deg(ei3, ew2):
    f32 = jnp.float32
    kern = pl.kernel(
        _deg_body,
        out_type=[jax.ShapeDtypeStruct((NC, NP), f32)],
        mesh=_sc_mesh(),
        compiler_params=pltpu.CompilerParams(needs_layout_passes=False),
        scratch_types=[
            pltpu.VMEM_SHARED((NP,), f32),
            pltpu.VMEM((CH, BB), jnp.int32),
            pltpu.VMEM((CH, BB), f32),
            pltpu.VMEM((BB,), f32),
            pltpu.VMEM((DVT,), f32),
            pltpu.SemaphoreType.DMA,
        ],
    )
    return kern(ei3, ew2)[0]


def _hop1(ei3, ew2, x):
    f32 = jnp.float32
    kern = pl.kernel(
        _hop1_body,
        out_type=[jax.ShapeDtypeStruct((NC, NP, FEAT), f32),
                  jax.ShapeDtypeStruct((TOTB, BB), f32)],
        mesh=_sc_mesh(),
        compiler_params=pltpu.CompilerParams(needs_layout_passes=False),
        scratch_types=[
            pltpu.VMEM_SHARED((NP, FEAT), f32),
            pltpu.VMEM_SHARED((NP,), f32),
            pltpu.VMEM((BB, FEAT), f32),
            pltpu.VMEM((BB, FEAT), f32),
            pltpu.VMEM((CH, BB), jnp.int32),
            pltpu.VMEM((CH, BB), jnp.int32),
            pltpu.VMEM((CH, BB), f32),
            pltpu.VMEM((CH, BB), f32),
            pltpu.VMEM((CH, BB), f32),
            pltpu.VMEM((DVT,), f32),
            pltpu.SemaphoreType.DMA,
            pltpu.SemaphoreType.DMA,
            pltpu.SemaphoreType.DMA,
            pltpu.SemaphoreType.DMA,
            pltpu.SemaphoreType.DMA,
        ],
    )
    return kern(ei3, ew2, x)


def _hop2(ei3, nrm2, tx1):
    f32 = jnp.float32
    kern = pl.kernel(
        _hop2_body,
        out_type=[jax.ShapeDtypeStruct((NC, NP, FEAT), f32)],
        mesh=_sc_mesh(),
        compiler_params=pltpu.CompilerParams(needs_layout_passes=False),
        scratch_types=[
            pltpu.VMEM_SHARED((NP, FEAT), f32),
            pltpu.VMEM((BB, FEAT), f32),
            pltpu.VMEM((BB, FEAT), f32),
            pltpu.VMEM((CH, BB), jnp.int32),
            pltpu.VMEM((CH, BB), jnp.int32),
            pltpu.VMEM((CH, BB), f32),
            pltpu.SemaphoreType.DMA,
            pltpu.SemaphoreType.DMA,
            pltpu.SemaphoreType.DMA,
            pltpu.SemaphoreType.DMA,
        ],
    )
    return kern(ei3, nrm2, tx1)[0]


def _dinv_body(degp_ref, out_ref):
    d = degp_ref[0] + degp_ref[1]
    out_ref[...] = jnp.where(d > 0, lax.rsqrt(jnp.maximum(d, 1e-12)), 0.0)


def _dinv(degp):
    return pl.pallas_call(
        _dinv_body,
        out_shape=jax.ShapeDtypeStruct((NP // FEAT, FEAT), jnp.float32),
    )(degp.reshape(NC, NP // FEAT, FEAT)).reshape(NP)


def _sum_body(p_ref, out_ref):
    out_ref[...] = p_ref[0] + p_ref[1]


def _sum_partials(p):
    R = 1024
    return pl.pallas_call(
        _sum_body,
        grid=(NP // R,),
        in_specs=[pl.BlockSpec((NC, R, FEAT), lambda i: (0, i, 0))],
        out_specs=pl.BlockSpec((R, FEAT), lambda i: (i, 0)),
        out_shape=jax.ShapeDtypeStruct((NP, FEAT), jnp.float32),
    )(p)


def _dense_body(x_ref, t1_ref, s2_ref, wzh_ref, bzh_ref, wlin_ref, blin_ref,
                out_ref):
    xb = x_ref[...]
    t1 = t1_ref[...]
    s2 = s2_ref[0] + s2_ref[1]
    tx2 = 2.0 * s2 - xb
    xt = jnp.concatenate([xb, t1, tx2], axis=1)
    a = jnp.dot(xt, wzh_ref[...], preferred_element_type=jnp.float32)
    a = a + bzh_ref[...]
    z = jax.nn.sigmoid(a[:, :FEAT])
    ht = jnp.tanh(a[:, FEAT:])
    h = jnp.maximum((1.0 - z) * ht, 0.0)
    out_ref[...] = (jnp.dot(h, wlin_ref[...], preferred_element_type=jnp.float32)
                    + blin_ref[...])


def _dense(x, tx1, s2p, wzh, bzh, wlin, blin):
    R = 512
    return pl.pallas_call(
        _dense_body,
        grid=(NP // R,),
        in_specs=[
            pl.BlockSpec((R, FEAT), lambda i: (i, 0)),
            pl.BlockSpec((R, FEAT), lambda i: (i, 0)),
            pl.BlockSpec((NC, R, FEAT), lambda i: (0, i, 0)),
            pl.BlockSpec((3 * FEAT, 2 * FEAT), lambda i: (0, 0)),
            pl.BlockSpec((1, 2 * FEAT), lambda i: (0, 0)),
            pl.BlockSpec((FEAT, FEAT), lambda i: (0, 0)),
            pl.BlockSpec((1, FEAT), lambda i: (0, 0)),
        ],
        out_specs=pl.BlockSpec((R, FEAT), lambda i: (i, 0)),
        out_shape=jax.ShapeDtypeStruct((N, FEAT), jnp.float32),
    )(x, tx1, s2p, wzh, bzh, wlin, blin)


def kernel(x, edge_index, edge_weight, Wxz, bxz, Whz, bhz, Wxr, bxr, Whr, bhr,
           Wxh, bxh, Whh, bhh, Wlin, blin):
    pad = EP - E
    ei3 = jnp.concatenate(
        [edge_index, jnp.zeros((2, pad), jnp.int32)], axis=1
    ).reshape(2, TOTB, BB)
    ew2 = jnp.concatenate(
        [edge_weight, jnp.zeros((pad,), jnp.float32)]).reshape(TOTB, BB)
    tx1p, nrm2 = _hop1(ei3, ew2, x)
    tx1 = _sum_partials(tx1p)
    s2p = _hop2(ei3, nrm2, tx1)
    wzh = jnp.concatenate([Wxz.reshape(3 * FEAT, FEAT),
                           Wxh.reshape(3 * FEAT, FEAT)], axis=1)
    bzh = jnp.concatenate([bxz + bhz, bxh + bhh]).reshape(1, 2 * FEAT)
    return _dense(x, tx1[:N], s2p[:, :N], wzh, bzh, Wlin, blin.reshape(1, FEAT))


# speedup vs baseline: 1.0773x; 1.0430x over previous
import jax
import jax.numpy as jnp
from jax import lax
from jax.experimental import pallas as pl
from jax.experimental.pallas import tpu as pltpu
from jax.experimental.pallas import tpu_sc as plsc

N = 10000
E = 320000
FEAT = 128
NP = 10240
NC = 2
NT = 16
NW = NC * NT
BB = 128
TOTB = E // BB
GROUPS = BB // 16
DVT = NP // NT
VREGS = FEAT // 16


def _rsqrt16(v):
    bits = lax.bitcast_convert_type(v, jnp.int32)
    y = lax.bitcast_convert_type(
        jnp.full((16,), 0x5F3759DF, jnp.int32) - (bits >> 1), jnp.float32)
    for _ in range(3):
        y = y * (1.5 - 0.5 * v * y * y)
    return y


def _splat(v16, j):
    return v16.at[jnp.full((16,), j, jnp.int32)].get(mode="promise_in_bounds")


def _scale_rows(rows, norms):
    for g in range(GROUPS):
        for j in range(16):
            sc = _splat(norms[g], j)
            e = g * 16 + j
            for q in range(VREGS):
                rows[e, pl.ds(q * 16, 16)] = rows[e, pl.ds(q * 16, 16)] * sc


def _zero_shared(acc_sh, zbuf, r0):
    z16 = jnp.zeros((16,), jnp.float32)
    for i in range(64):
        for q in range(VREGS):
            zbuf[i, pl.ds(q * 16, 16)] = z16
    for i in range(NP // NT // 64):
        pltpu.sync_copy(zbuf, acc_sh.at[pl.ds(r0 + i * 64, 64)])


def _hop1_body(ei, ew, x, tx1_out, norm_out,
               tx1_sh, deg_sh, dinv_sh, dinv_loc, rows, srcb, dstb, ewb,
               normb, degloc, zbuf, sem):
    c = lax.axis_index("c")
    s = lax.axis_index("s")
    wid = s * NC + c
    r0 = s * (NP // NT)

    _zero_shared(tx1_sh, zbuf, r0)
    z16 = jnp.zeros((16,), jnp.float32)
    for i in range(DVT // 16):
        degloc[pl.ds(i * 16, 16)] = z16
    pltpu.sync_copy(degloc, deg_sh.at[pl.ds(s * DVT, DVT)])
    plsc.subcore_barrier()

    def deg_body(k, carry):
        off = (s + k * NT) * BB
        pltpu.sync_copy(ew.at[pl.ds(off, BB)], ewb)
        pltpu.sync_copy(ei.at[0].at[pl.ds(off, BB)], srcb)
        pltpu.sync_copy(ewb, deg_sh.at[srcb], add=True)
        return carry

    lax.fori_loop(0, (TOTB - s + NT - 1) // NT, deg_body, 0)
    plsc.subcore_barrier()

    pltpu.sync_copy(deg_sh.at[pl.ds(s * DVT, DVT)], degloc)
    for i in range(DVT // 16):
        d16 = degloc[pl.ds(i * 16, 16)]
        y = _rsqrt16(jnp.maximum(d16, 1e-12))
        degloc[pl.ds(i * 16, 16)] = jnp.where(d16 > 0, y, 0.0)
    pltpu.sync_copy(degloc, dinv_sh.at[pl.ds(s * DVT, DVT)])
    plsc.subcore_barrier()
    pltpu.sync_copy(dinv_sh, dinv_loc)

    def hop_body(k, carry):
        off = (wid + k * NW) * BB
        pltpu.sync_copy(ei.at[0].at[pl.ds(off, BB)], srcb)
        pltpu.sync_copy(ei.at[1].at[pl.ds(off, BB)], dstb)
        pltpu.sync_copy(ew.at[pl.ds(off, BB)], ewb)
        norms = []
        for g in range(GROUPS):
            s16 = srcb[pl.ds(g * 16, 16)]
            d16 = dstb[pl.ds(g * 16, 16)]
            e16 = ewb[pl.ds(g * 16, 16)]
            dv_s = plsc.load_gather(dinv_loc, [s16])
            dv_d = plsc.load_gather(dinv_loc, [d16])
            n16 = -(dv_s * e16 * dv_d)
            normb[pl.ds(g * 16, 16)] = n16
            norms.append(n16)
        pltpu.sync_copy(normb, norm_out.at[pl.ds(off, BB)])
        pltpu.async_copy(x.at[srcb], rows, sem).wait()
        _scale_rows(rows, norms)
        pltpu.sync_copy(rows, tx1_sh.at[dstb], add=True)
        return carry

    lax.fori_loop(0, (TOTB - wid + NW - 1) // NW, hop_body, 0)
    plsc.subcore_barrier()

    pltpu.sync_copy(tx1_sh.at[pl.ds(r0, NP // NT)],
                    tx1_out.at[c].at[pl.ds(r0, NP // NT)])


def _hop2_body(ei, nrm, tx1, s2_out,
               s2_sh, rows, srcb, dstb, normb, zbuf, sem):
    c = lax.axis_index("c")
    s = lax.axis_index("s")
    wid = s * NC + c
    r0 = s * (NP // NT)

    _zero_shared(s2_sh, zbuf, r0)
    plsc.subcore_barrier()

    def hop_body(k, carry):
        off = (wid + k * NW) * BB
        pltpu.sync_copy(ei.at[0].at[pl.ds(off, BB)], srcb)
        pltpu.sync_copy(ei.at[1].at[pl.ds(off, BB)], dstb)
        pltpu.sync_copy(nrm.at[pl.ds(off, BB)], normb)
        norms = [normb[pl.ds(g * 16, 16)] for g in range(GROUPS)]
        pltpu.async_copy(tx1.at[srcb], rows, sem).wait()
        _scale_rows(rows, norms)
        pltpu.sync_copy(rows, s2_sh.at[dstb], add=True)
        return carry

    lax.fori_loop(0, (TOTB - wid + NW - 1) // NW, hop_body, 0)
    plsc.subcore_barrier()

    pltpu.sync_copy(s2_sh.at[pl.ds(r0, NP // NT)],
                    s2_out.at[c].at[pl.ds(r0, NP // NT)])


def _sc_mesh():
    return plsc.VectorSubcoreMesh(core_axis_name="c", subcore_axis_name="s")


def _hop1(ei, ew, x):
    f32 = jnp.float32
    kern = pl.kernel(
        _hop1_body,
        out_type=[jax.ShapeDtypeStruct((NC, NP, FEAT), f32),
                  jax.ShapeDtypeStruct((E,), f32)],
        mesh=_sc_mesh(),
        compiler_params=pltpu.CompilerParams(needs_layout_passes=False),
        scratch_types=[
            pltpu.VMEM_SHARED((NP, FEAT), f32),
            pltpu.VMEM_SHARED((NP,), f32),
            pltpu.VMEM_SHARED((NP,), f32),
            pltpu.VMEM((NP,), f32),
            pltpu.VMEM((BB, FEAT), f32),
            pltpu.VMEM((BB,), jnp.int32),
            pltpu.VMEM((BB,), jnp.int32),
            pltpu.VMEM((BB,), f32),
            pltpu.VMEM((BB,), f32),
            pltpu.VMEM((DVT,), f32),
            pltpu.VMEM((64, FEAT), f32),
            pltpu.SemaphoreType.DMA,
        ],
    )
    return kern(ei, ew, x)


def _hop2(ei, nrm, tx1):
    f32 = jnp.float32
    kern = pl.kernel(
        _hop2_body,
        out_type=[jax.ShapeDtypeStruct((NC, NP, FEAT), f32)],
        mesh=_sc_mesh(),
        compiler_params=pltpu.CompilerParams(needs_layout_passes=False),
        scratch_types=[
            pltpu.VMEM_SHARED((NP, FEAT), f32),
            pltpu.VMEM((BB, FEAT), f32),
            pltpu.VMEM((BB,), jnp.int32),
            pltpu.VMEM((BB,), jnp.int32),
            pltpu.VMEM((BB,), f32),
            pltpu.VMEM((64, FEAT), f32),
            pltpu.SemaphoreType.DMA,
        ],
    )
    return kern(ei, nrm, tx1)[0]


def _sum_body(p_ref, out_ref):
    out_ref[...] = p_ref[0] + p_ref[1]


def _sum_partials(p):
    R = 1024
    return pl.pallas_call(
        _sum_body,
        grid=(NP // R,),
        in_specs=[pl.BlockSpec((NC, R, FEAT), lambda i: (0, i, 0))],
        out_specs=pl.BlockSpec((R, FEAT), lambda i: (i, 0)),
        out_shape=jax.ShapeDtypeStruct((NP, FEAT), jnp.float32),
    )(p)


def _dense_body(x_ref, t1_ref, s2_ref, wzh_ref, bzh_ref, wlin_ref, blin_ref,
                out_ref):
    xb = x_ref[...]
    t1 = t1_ref[...]
    s2 = s2_ref[0] + s2_ref[1]
    tx2 = 2.0 * s2 - xb
    xt = jnp.concatenate([xb, t1, tx2], axis=1)
    a = jnp.dot(xt, wzh_ref[...], preferred_element_type=jnp.float32)
    a = a + bzh_ref[...]
    z = jax.nn.sigmoid(a[:, :FEAT])
    ht = jnp.tanh(a[:, FEAT:])
    h = jnp.maximum((1.0 - z) * ht, 0.0)
    out_ref[...] = (jnp.dot(h, wlin_ref[...], preferred_element_type=jnp.float32)
                    + blin_ref[...])


def _dense(x, tx1, s2p, wzh, bzh, wlin, blin):
    R = 512
    return pl.pallas_call(
        _dense_body,
        grid=(NP // R,),
        in_specs=[
            pl.BlockSpec((R, FEAT), lambda i: (i, 0)),
            pl.BlockSpec((R, FEAT), lambda i: (i, 0)),
            pl.BlockSpec((NC, R, FEAT), lambda i: (0, i, 0)),
            pl.BlockSpec((3 * FEAT, 2 * FEAT), lambda i: (0, 0)),
            pl.BlockSpec((1, 2 * FEAT), lambda i: (0, 0)),
            pl.BlockSpec((FEAT, FEAT), lambda i: (0, 0)),
            pl.BlockSpec((1, FEAT), lambda i: (0, 0)),
        ],
        out_specs=pl.BlockSpec((R, FEAT), lambda i: (i, 0)),
        out_shape=jax.ShapeDtypeStruct((N, FEAT), jnp.float32),
    )(x, tx1, s2p, wzh, bzh, wlin, blin)


def kernel(x, edge_index, edge_weight, Wxz, bxz, Whz, bhz, Wxr, bxr, Whr, bhr,
           Wxh, bxh, Whh, bhh, Wlin, blin):
    tx1p, nrm = _hop1(edge_index, edge_weight, x)
    tx1 = _sum_partials(tx1p)
    s2p = _hop2(edge_index, nrm, tx1)
    wzh = jnp.concatenate([Wxz.reshape(3 * FEAT, FEAT),
                           Wxh.reshape(3 * FEAT, FEAT)], axis=1)
    bzh = jnp.concatenate([bxz + bhz, bxh + bhh]).reshape(1, 2 * FEAT)
    return _dense(x, tx1[:N], s2p[:, :N], wzh, bzh, Wlin, blin.reshape(1, FEAT))
